# baseline (speedup 1.0000x reference)
import functools

import jax
import jax.numpy as jnp
from jax.experimental import pallas as pl


def _loss_body(pred_ref, out_ref):
    x = pred_ref[:, 0] - pred_ref[:, 1]
    out_ref[...] = jnp.full((1, 1), jnp.mean(jnp.log1p(jnp.exp(-x))))


def _propagate(featA, featB, src, dst, n_layers):
    nA, nB = featA.shape[0], featB.shape[0]
    N = nA + nB
    r = jnp.concatenate([src, dst + nA])
    c = jnp.concatenate([dst + nA, src])
    deg = jnp.bincount(r, length=N).astype(jnp.float32)
    norm = 1.0 / (jnp.sqrt(deg) + 1e-8)
    v = norm[r] * norm[c]
    feat = jnp.concatenate([featA, featB], axis=0)
    feats = [feat]
    for _ in range(n_layers):
        msg = v[:, None] * feat[c]
        feat = jax.ops.segment_sum(msg, r, num_segments=N)
        feats.append(feat)
    feat = jnp.mean(jnp.stack(feats, axis=0), axis=0)
    return feat[:nA], feat[nA:]


def kernel(users_feature, items_feature, bundles_feature, W1, b1, W2, b2,
           ui_u, ui_i, ub_u, ub_b, bundle_item_ids, bundle_item_mask,
           users, bundles):
    IL_users, IL_items = _propagate(users_feature, items_feature, ui_u, ui_i, 2)
    BL_users, BL_bundles = _propagate(users_feature, bundles_feature, ub_u, ub_b, 2)

    u = users
    b = bundles
    IL_u = IL_users[u]
    item_ids = bundle_item_ids[b]
    item_emb = IL_items[item_ids]
    att_logits = jnp.einsum('bd,bnsd->bns', IL_u, item_emb)
    mask = bundle_item_mask[b]
    att_logits = jnp.where(mask, att_logits, -1e9)
    att = jax.nn.softmax(att_logits, axis=-1)
    IL_b = jnp.einsum('bns,bnsd->bnd', att, item_emb)

    BL_u = BL_users[u]
    BL_b = BL_bundles[b]

    gate_in = jnp.concatenate([BL_u, IL_u], axis=-1)
    g = jax.nn.sigmoid(jax.nn.relu(gate_in @ W1 + b1) @ W2 + b2)
    user_rep = g * BL_u + (1.0 - g) * IL_u
    bund_rep = g[:, None, :] * BL_b + (1.0 - g)[:, None, :] * IL_b
    pred = jnp.sum(user_rep[:, None, :] * bund_rep, axis=-1)

    loss = pl.pallas_call(
        _loss_body,
        out_shape=jax.ShapeDtypeStruct((1, 1), jnp.float32),
    )(pred)
    return loss.reshape(())


# speedup vs baseline: 11.3394x; 11.3394x over previous
import functools

import jax
import jax.numpy as jnp
from jax import lax
from jax.experimental import pallas as pl
from jax.experimental.pallas import tpu as pltpu
from jax.experimental.pallas import tpu_sc as plsc

D = 64
NU, NI, NBD = 50000, 100000, 20000
PADU, PADI, PADB = 50688, 100480, 20096
CH_U = 16896
CH_I = 20096
DUM = 256
ACC_ROWS = CH_I + DUM
EB = 512
E_UI, E_UB = 1_000_000, 500_000
EP_UI, EP_UB = 1_015_808, 507_904
BATCH = 1024
S_ITEMS = 12

OFF_UI_A, OFF_UI_B = 0, PADU
OFF_UB_A, OFF_UB_B = PADU + PADI, PADU + PADI + PADU
DEGTOT = PADU + PADI + PADU + PADB
DEGA = 229376

@functools.cache
def _mesh():
    return plsc.VectorSubcoreMesh(core_axis_name="c", subcore_axis_name="s",
                                  num_cores=2, num_subcores=16)


def _deg_body(sui, dui, sub, dub, degp, ones1, idxb, idx2d, stage, acc, sem):
    c = lax.axis_index("c")
    s = lax.axis_index("s")
    wid = s * 2 + c
    for dd in range(8):
        ones1[pl.ds(dd * 16, 16)] = jnp.ones((16,), jnp.float32)
    for dd in range(64):
        stage[pl.ds(dd * 16, 16)] = jnp.zeros((16,), jnp.float32)
    pt = DEGA // 16
    for u in range(14):
        pltpu.sync_copy(stage, acc.at[pl.ds(s * pt + u * 1024, 1024)])
    plsc.subcore_barrier()

    def scan_list(eref, off, nblk):
        def body(k, carry):
            blk = k * 32 + wid
            pltpu.sync_copy(eref.at[pl.ds(blk * EB, EB)], idxb)
            for j in range(4):
                for t in range(8):
                    v = idxb[pl.ds(j * 128 + t * 16, 16)] + off
                    idx2d[j, pl.ds(t * 16, 16)] = v
            for j in range(4):
                pltpu.sync_copy(ones1, acc.at[idx2d.at[j]], add=True)
            return carry
        lax.fori_loop(0, nblk // 32, body, 0)

    scan_list(sui, OFF_UI_A, EP_UI // EB)
    scan_list(dui, OFF_UI_B, EP_UI // EB)
    scan_list(sub, OFF_UB_A, EP_UB // EB)
    scan_list(dub, OFF_UB_B, EP_UB // EB)
    plsc.subcore_barrier()
    for u in range(14):
        pltpu.sync_copy(acc.at[pl.ds(s * pt + u * 1024, 1024)], stage)
        pltpu.sync_copy(stage, degp.at[pl.ds(c * DEGA + s * pt + u * 1024,
                                             1024)])


def _deg_call(sui, dui, sub, dub):
    k = pl.kernel(
        _deg_body,
        out_type=jax.ShapeDtypeStruct((2 * DEGA,), jnp.float32),
        mesh=_mesh(),
        scratch_types=[
            pltpu.VMEM((128,), jnp.float32),
            pltpu.VMEM((EB,), jnp.int32),
            pltpu.VMEM((4, 128), jnp.int32),
            pltpu.VMEM((1024,), jnp.float32),
            pltpu.VMEM_SHARED((DEGA,), jnp.float32),
            pltpu.SemaphoreType.DMA,
        ],
        compiler_params=pltpu.CompilerParams(use_tc_tiling_on_sc=False, needs_layout_passes=False),
    )
    return k(sui, dui, sub, dub)


def _make_layer(chunks, nblk):
    def body(src, dst, ha, hb, sa, sb, sidx, didx, idx2d, rows, zbuf, acc, sem):
        c = lax.axis_index("c")
        s = lax.axis_index("s")
        for dd in range(32):
            zbuf[dd // 4, pl.ds((dd % 4) * 16, 16)] = jnp.zeros((16,),
                                                               jnp.float32)
        for (side, base, size, asn) in chunks:

            @pl.when(c == asn)
            def _(side=side, base=base, size=size):
                gsrc = hb if side == 0 else ha
                out = sa if side == 0 else sb
                rpt = (size + DUM) // 16

                def zloop(u, carry):
                    pltpu.sync_copy(zbuf, acc.at[pl.ds(s * rpt + u * 8, 8)])
                    return carry

                lax.fori_loop(0, rpt // 8, zloop, 0)
                plsc.subcore_barrier()

                def eblk(k, carry):
                    blk = k * 16 + s
                    off = blk * EB
                    pltpu.sync_copy(src.at[pl.ds(off, EB)], sidx)
                    pltpu.sync_copy(dst.at[pl.ds(off, EB)], didx)
                    ridx = sidx if side == 0 else didx
                    gidx = didx if side == 0 else sidx
                    for j in range(4):
                        for t in range(8):
                            rv = ridx[pl.ds(j * 128 + t * 16, 16)]
                            ok = (rv >= base) & (rv < base + size)
                            lv = jnp.where(ok, rv - base,
                                           size + (rv & (DUM - 1)))
                            idx2d[j, pl.ds(t * 16, 16)] = lv
                    cps = [pltpu.async_copy(
                        gsrc.at[gidx.at[pl.ds(j * 128, 128)]],
                        rows.at[pl.ds(j * 128, 128)], sem)
                        for j in range(4)]
                    for cp in cps:
                        cp.wait()
                    for j in range(4):
                        pltpu.sync_copy(rows.at[pl.ds(j * 128, 128)],
                                        acc.at[idx2d.at[j]], add=True)
                    return carry

                lax.fori_loop(0, nblk // 16, eblk, 0)
                plsc.subcore_barrier()
                dpt = size // 16
                for off in range(0, dpt, EB):
                    sz = min(EB, dpt - off)
                    pltpu.sync_copy(acc.at[pl.ds(s * dpt + off, sz)],
                                    rows.at[pl.ds(0, sz)])
                    pltpu.sync_copy(rows.at[pl.ds(0, sz)],
                                    out.at[pl.ds(base + s * dpt + off, sz)])

    return body


def _layer_call(chunks, nblk, na_pad, nb_pad, src, dst, ha, hb):
    body = _make_layer(chunks, nblk)
    k = pl.kernel(
        body,
        out_type=(jax.ShapeDtypeStruct((na_pad, D), jnp.float32),
                  jax.ShapeDtypeStruct((nb_pad, D), jnp.float32)),
        mesh=_mesh(),
        scratch_types=[
            pltpu.VMEM((EB,), jnp.int32),
            pltpu.VMEM((EB,), jnp.int32),
            pltpu.VMEM((4, 128), jnp.int32),
            pltpu.VMEM((EB, D), jnp.float32),
            pltpu.VMEM((8, D), jnp.float32),
            pltpu.VMEM_SHARED((ACC_ROWS, D), jnp.float32),
            pltpu.SemaphoreType.DMA,
        ],
        compiler_params=pltpu.CompilerParams(use_tc_tiling_on_sc=False, needs_layout_passes=False),
    )
    return k(src, dst, ha, hb)


UI_CHUNKS = ((0, 0, CH_U, 0), (0, CH_U, CH_U, 1), (0, 2 * CH_U, CH_U, 0),
             (1, 0, CH_I, 1), (1, CH_I, CH_I, 0), (1, 2 * CH_I, CH_I, 1),
             (1, 3 * CH_I, CH_I, 0), (1, 4 * CH_I, CH_I, 1))
UB_CHUNKS = ((0, 0, CH_U, 0), (0, CH_U, CH_U, 1), (0, 2 * CH_U, CH_U, 1),
             (1, 0, PADB, 0))


def _gather_body(uix_h, bfl_h, bit_h, ilu_t, ili_t, blu_t, blb_t,
                 ilu_g, blu_g, ie_g, blb_g,
                 uix, bflv, uix32, bix64, ids2d, iix, rows, sem):
    c = lax.axis_index("c")
    s = lax.axis_index("s")
    wid = s * 2 + c
    pltpu.sync_copy(uix_h, uix)
    pltpu.sync_copy(bfl_h, bflv)
    for t in range(2):
        uix32[pl.ds(t * 16, 16)] = uix[pl.ds(wid * 32 + t * 16, 16)]
    for t in range(4):
        bix64[pl.ds(t * 16, 16)] = bflv[pl.ds(wid * 64 + t * 16, 16)]
    pltpu.async_copy(ilu_t.at[uix32], rows.at[pl.ds(0, 32)], sem).wait()
    pltpu.sync_copy(rows.at[pl.ds(0, 32)], ilu_g.at[pl.ds(wid * 32, 32)])
    pltpu.async_copy(blu_t.at[uix32], rows.at[pl.ds(0, 32)], sem).wait()
    pltpu.sync_copy(rows.at[pl.ds(0, 32)], blu_g.at[pl.ds(wid * 32, 32)])
    pltpu.async_copy(blb_t.at[bix64], rows.at[pl.ds(0, 64)], sem).wait()
    pltpu.sync_copy(rows.at[pl.ds(0, 64)], blb_g.at[pl.ds(wid * 64, 64)])
    for j in range(16):
        pltpu.async_copy(bit_h.at[bflv.at[pl.ds(j * 128, 128)]],
                         ids2d.at[pl.ds(j * 128, 128)], sem).wait()
    iota = lax.iota(jnp.int32, 16)
    for i in range(48):
        row0 = wid * 768 + i * 16
        kk = row0 // BATCH
        n = kk // S_ITEMS
        si = kk % S_ITEMS
        b0 = row0 % BATCH
        rvec = (n * BATCH + b0) + iota
        cvec = jnp.full((16,), si, jnp.int32)
        iv = plsc.load_gather(ids2d, [rvec, cvec])
        iix[pl.ds(i * 16, 16)] = iv
    for j in range(6):
        pltpu.async_copy(ili_t.at[iix.at[pl.ds(j * 128, 128)]],
                         rows.at[pl.ds(j * 128, 128)], sem).wait()
    pltpu.sync_copy(rows, ie_g.at[pl.ds(wid * 768, 768)])


def _gather_call(uix, bfl, bit, ilu_t, ili_t, blu_t, blb_t):
    k = pl.kernel(
        _gather_body,
        out_type=(jax.ShapeDtypeStruct((BATCH, D), jnp.float32),
                  jax.ShapeDtypeStruct((BATCH, D), jnp.float32),
                  jax.ShapeDtypeStruct((24576, D), jnp.float32),
                  jax.ShapeDtypeStruct((2048, D), jnp.float32)),
        mesh=_mesh(),
        scratch_types=[
            pltpu.VMEM((BATCH,), jnp.int32),
            pltpu.VMEM((2048,), jnp.int32),
            pltpu.VMEM((32,), jnp.int32),
            pltpu.VMEM((64,), jnp.int32),
            pltpu.VMEM((2048, 16), jnp.int32),
            pltpu.VMEM((768,), jnp.int32),
            pltpu.VMEM((768, D), jnp.float32),
            pltpu.SemaphoreType.DMA,
        ],
        compiler_params=pltpu.CompilerParams(use_tc_tiling_on_sc=False, needs_layout_passes=False),
    )
    return k(uix, bfl, bit, ilu_t, ili_t, blu_t, blb_t)


def _norm_body(degp_ref, out_ref):
    deg = degp_ref[pl.ds(0, DEGA)] + degp_ref[pl.ds(DEGA, DEGA)]
    out_ref[...] = 1.0 / (jnp.sqrt(deg) + 1e-8)


def _norm_call(degp):
    return pl.pallas_call(
        _norm_body,
        out_shape=jax.ShapeDtypeStruct((DEGA,), jnp.float32),
    )(degp)


def _scale_body(p, x_ref, nv_ref, o_ref):
    n = nv_ref[...]
    if p == 2:
        n = n * n
    o_ref[...] = x_ref[...] * n


def _scale(x, nv, p):
    npad = x.shape[0]
    rb = npad // 16
    return pl.pallas_call(
        functools.partial(_scale_body, p),
        grid=(16,),
        in_specs=[pl.BlockSpec((rb, D), lambda i: (i, 0)),
                  pl.BlockSpec((rb, 1), lambda i: (i, 0))],
        out_specs=pl.BlockSpec((rb, D), lambda i: (i, 0)),
        out_shape=jax.ShapeDtypeStruct((npad, D), jnp.float32),
    )(x, nv)


def _combine_body(f0_ref, s1_ref, s2_ref, nv_ref, o_ref):
    n = nv_ref[...]
    o_ref[...] = (f0_ref[...] + (s1_ref[...] + s2_ref[...]) * n) / 3.0


def _combine(f0, s1, s2, nv):
    npad = f0.shape[0]
    rb = npad // 16
    return pl.pallas_call(
        _combine_body,
        grid=(16,),
        in_specs=[pl.BlockSpec((rb, D), lambda i: (i, 0)),
                  pl.BlockSpec((rb, D), lambda i: (i, 0)),
                  pl.BlockSpec((rb, D), lambda i: (i, 0)),
                  pl.BlockSpec((rb, 1), lambda i: (i, 0))],
        out_specs=pl.BlockSpec((rb, D), lambda i: (i, 0)),
        out_shape=jax.ShapeDtypeStruct((npad, D), jnp.float32),
    )(f0, s1, s2, nv)


def _head_body(ilu_ref, blu_ref, ie_ref, blb_ref, w1_ref, b1_ref, w2_ref,
               b2_ref, out_ref):
    il_u = ilu_ref[...]
    bl_u = blu_ref[...]
    il_b = []
    for n in range(2):
        es = [ie_ref[pl.ds((n * S_ITEMS + si) * BATCH, BATCH), :]
              for si in range(S_ITEMS)]
        ls = [jnp.sum(il_u * e, axis=1) for e in es]
        m = ls[0]
        for l in ls[1:]:
            m = jnp.maximum(m, l)
        ws = [jnp.exp(l - m) for l in ls]
        z = ws[0]
        for w in ws[1:]:
            z = z + w
        acc = ws[0][:, None] * es[0]
        for w, e in zip(ws[1:], es[1:]):
            acc = acc + w[:, None] * e
        il_b.append(acc / z[:, None])

    gate_in = jnp.concatenate([bl_u, il_u], axis=1)
    h = jnp.maximum(jnp.dot(gate_in, w1_ref[...],
                            preferred_element_type=jnp.float32)
                    + b1_ref[...][None, :], 0.0)
    g = jax.nn.sigmoid(jnp.dot(h, w2_ref[...],
                               preferred_element_type=jnp.float32)
                       + b2_ref[...][None, :])
    user_rep = g * bl_u + (1.0 - g) * il_u
    preds = []
    for n in range(2):
        bl_b = blb_ref[pl.ds(n * BATCH, BATCH), :]
        bund = g * bl_b + (1.0 - g) * il_b[n]
        preds.append(jnp.sum(user_rep * bund, axis=1))
    x = preds[0] - preds[1]
    out_ref[...] = jnp.full((1, 1), jnp.mean(jnp.log1p(jnp.exp(-x))))


def _head_call(ilu, blu, ie, blb, w1, b1, w2, b2):
    return pl.pallas_call(
        _head_body,
        out_shape=jax.ShapeDtypeStruct((1, 1), jnp.float32),
    )(ilu, blu, ie, blb, w1, b1, w2, b2)


def _pad_rows(x, n_pad):
    return jnp.pad(x, ((0, n_pad - x.shape[0]), (0, 0)))


def _pad_edges(e, e_pad, base, span):
    extra = base + (jnp.arange(e_pad - e.shape[0], dtype=jnp.int32) % span)
    return jnp.concatenate([e, extra])


def kernel(users_feature, items_feature, bundles_feature, W1, b1, W2, b2,
           ui_u, ui_i, ub_u, ub_b, bundle_item_ids, bundle_item_mask,
           users, bundles):
    f0u = _pad_rows(users_feature, PADU)
    f0i = _pad_rows(items_feature, PADI)
    f0b = _pad_rows(bundles_feature, PADB)

    sui = _pad_edges(ui_u.astype(jnp.int32), EP_UI, NU, PADU - NU)
    dui = _pad_edges(ui_i.astype(jnp.int32), EP_UI, NI, PADI - NI)
    sub = _pad_edges(ub_u.astype(jnp.int32), EP_UB, NU, PADU - NU)
    dub = _pad_edges(ub_b.astype(jnp.int32), EP_UB, NBD, PADB - NBD)

    degp = _deg_call(sui, dui, sub, dub)
    norm = _norm_call(degp)
    n_ui_a = norm[OFF_UI_A:OFF_UI_A + PADU, None]
    n_ui_b = norm[OFF_UI_B:OFF_UI_B + PADI, None]
    n_ub_a = norm[OFF_UB_A:OFF_UB_A + PADU, None]
    n_ub_b = norm[OFF_UB_B:OFF_UB_B + PADB, None]

    def propagate(srcp, dstp, f0a, f0b_, nva, nvb, chunks, nblk, napad, nbpad):
        h0a = _scale(f0a, nva, 1)
        h0b = _scale(f0b_, nvb, 1)
        s1a, s1b = _layer_call(chunks, nblk, napad, nbpad, srcp, dstp,
                               h0a, h0b)
        h1a = _scale(s1a, nva, 2)
        h1b = _scale(s1b, nvb, 2)
        s2a, s2b = _layer_call(chunks, nblk, napad, nbpad, srcp, dstp,
                               h1a, h1b)
        outa = _combine(f0a, s1a, s2a, nva)
        outb = _combine(f0b_, s1b, s2b, nvb)
        return outa, outb

    il_users, il_items = propagate(sui, dui, f0u, f0i, n_ui_a, n_ui_b,
                                   UI_CHUNKS, EP_UI // EB, PADU, PADI)
    bl_users, bl_bundles = propagate(sub, dub, f0u, f0b, n_ub_a, n_ub_b,
                                     UB_CHUNKS, EP_UB // EB, PADU, PADB)

    bfl = jnp.transpose(bundles.astype(jnp.int32)).reshape(-1)
    bit16 = jnp.pad(bundle_item_ids.astype(jnp.int32), ((0, 0), (0, 4)))
    ilu_g, blu_g, ie_g, blb_g = _gather_call(
        users.astype(jnp.int32), bfl, bit16,
        il_users, il_items, bl_users, bl_bundles)

    loss = _head_call(ilu_g, blu_g, ie_g, blb_g, W1, b1, W2, b2)
    return loss.reshape(())


# speedup vs baseline: 16.2560x; 1.4336x over previous
import functools

import jax
import jax.numpy as jnp
from jax import lax
from jax.experimental import pallas as pl
from jax.experimental.pallas import tpu as pltpu
from jax.experimental.pallas import tpu_sc as plsc

D = 64
NU, NI, NBD = 50000, 100000, 20000
PADU, PADI, PADB = 50688, 100480, 20096
CH_U = 16896
CH_I = 20096
DUM = 256
ACC_ROWS = CH_I + DUM
EB = 256
E_UI, E_UB = 1_000_000, 500_000
EP_UI, EP_UB = 1_007_616, 507_904
BATCH = 1024
S_ITEMS = 12

OFF_UI_A, OFF_UI_B = 0, PADU
OFF_UB_A, OFF_UB_B = PADU + PADI, PADU + PADI + PADU
DEGTOT = PADU + PADI + PADU + PADB
DEGA = 229376

@functools.cache
def _mesh():
    return plsc.VectorSubcoreMesh(core_axis_name="c", subcore_axis_name="s",
                                  num_cores=2, num_subcores=16)


def _deg_body(sui, dui, sub, dub, degp, ones1, idxb, idx2d, stage, acc, sem):
    c = lax.axis_index("c")
    s = lax.axis_index("s")
    wid = s * 2 + c
    for dd in range(8):
        ones1[pl.ds(dd * 16, 16)] = jnp.ones((16,), jnp.float32)
    for dd in range(64):
        stage[pl.ds(dd * 16, 16)] = jnp.zeros((16,), jnp.float32)
    pt = DEGA // 16
    for u in range(14):
        pltpu.sync_copy(stage, acc.at[pl.ds(s * pt + u * 1024, 1024)])
    plsc.subcore_barrier()

    def scan_list(eref, off, nblk):
        def body(k, carry):
            blk = k * 32 + wid
            pltpu.sync_copy(eref.at[pl.ds(blk * EB, EB)], idxb)
            for j in range(2):
                for t in range(8):
                    v = idxb[pl.ds(j * 128 + t * 16, 16)] + off
                    idx2d[j, pl.ds(t * 16, 16)] = v
            for j in range(2):
                pltpu.sync_copy(ones1, acc.at[idx2d.at[j]], add=True)
            return carry
        lax.fori_loop(0, nblk // 32, body, 0)

    scan_list(sui, OFF_UI_A, EP_UI // EB)
    scan_list(dui, OFF_UI_B, EP_UI // EB)
    scan_list(sub, OFF_UB_A, EP_UB // EB)
    scan_list(dub, OFF_UB_B, EP_UB // EB)
    plsc.subcore_barrier()
    for u in range(14):
        pltpu.sync_copy(acc.at[pl.ds(s * pt + u * 1024, 1024)], stage)
        pltpu.sync_copy(stage, degp.at[pl.ds(c * DEGA + s * pt + u * 1024,
                                             1024)])


def _deg_call(sui, dui, sub, dub):
    k = pl.kernel(
        _deg_body,
        out_type=jax.ShapeDtypeStruct((2 * DEGA,), jnp.float32),
        mesh=_mesh(),
        scratch_types=[
            pltpu.VMEM((128,), jnp.float32),
            pltpu.VMEM((EB,), jnp.int32),
            pltpu.VMEM((2, 128), jnp.int32),
            pltpu.VMEM((1024,), jnp.float32),
            pltpu.VMEM_SHARED((DEGA,), jnp.float32),
            pltpu.SemaphoreType.DMA,
        ],
        compiler_params=pltpu.CompilerParams(use_tc_tiling_on_sc=False, needs_layout_passes=False),
    )
    return k(sui, dui, sub, dub)


def _make_layer(chunks, nblk):
    nb = nblk // 16

    def body(src, dst, ha, hb, sa, sb,
             sidx0, didx0, idx0, rows0, sidx1, didx1, idx1, rows1,
             zbuf, acc, semi0, semi1, semg0, semg1):
        c = lax.axis_index("c")
        s = lax.axis_index("s")
        bufs = ((sidx0, didx0, idx0, rows0, semi0, semg0),
                (sidx1, didx1, idx1, rows1, semi1, semg1))
        for dd in range(32):
            zbuf[dd // 4, pl.ds((dd % 4) * 16, 16)] = jnp.zeros((16,),
                                                               jnp.float32)

        for (side, base, size, asn) in chunks:

            @pl.when(c == asn)
            def _(side=side, base=base, size=size):
                gsrc = hb if side == 0 else ha
                out = sa if side == 0 else sb
                rpt = (size + DUM) // 16

                def zloop(u, carry):
                    pltpu.sync_copy(zbuf, acc.at[pl.ds(s * rpt + u * 8, 8)])
                    return carry

                lax.fori_loop(0, rpt // 8, zloop, 0)
                plsc.subcore_barrier()

                def fire_idx(i, bf):
                    off = (i * 16 + s) * EB
                    pltpu.async_copy(src.at[pl.ds(off, EB)], bf[0], bf[4])
                    pltpu.async_copy(dst.at[pl.ds(off, EB)], bf[1], bf[4])

                def drain_idx(bf):
                    pltpu.make_async_copy(src.at[pl.ds(0, EB)], bf[0],
                                          bf[4]).wait()
                    pltpu.make_async_copy(dst.at[pl.ds(0, EB)], bf[1],
                                          bf[4]).wait()

                def build_fire(bf):
                    ridx = bf[0] if side == 0 else bf[1]
                    gidx = bf[1] if side == 0 else bf[0]
                    for j in range(2):
                        for t in range(8):
                            rv = ridx[pl.ds(j * 128 + t * 16, 16)]
                            ok = (rv >= base) & (rv < base + size)
                            lv = jnp.where(ok, rv - base,
                                           size + (rv & (DUM - 1)))
                            bf[2][j, pl.ds(t * 16, 16)] = lv
                        pltpu.async_copy(
                            gsrc.at[gidx.at[pl.ds(j * 128, 128)]],
                            bf[3].at[pl.ds(j * 128, 128)], bf[5])

                def drain_scatter(bf):
                    pltpu.make_async_copy(gsrc.at[pl.ds(0, EB)], bf[3],
                                          bf[5]).wait()
                    for j in range(2):
                        pltpu.sync_copy(bf[3].at[pl.ds(j * 128, 128)],
                                        acc.at[bf[2].at[j]], add=True)

                fire_idx(0, bufs[0])
                fire_idx(1, bufs[1])
                drain_idx(bufs[0])
                build_fire(bufs[0])

                def pair(k2, carry):
                    for half in range(2):
                        i = 2 * k2 + half
                        bf = bufs[half]
                        ob = bufs[1 - half]

                        @pl.when(i + 1 < nb)
                        def _():
                            drain_idx(ob)
                            build_fire(ob)

                        drain_scatter(bf)

                        @pl.when(i + 2 < nb)
                        def _():
                            fire_idx(i + 2, bf)
                    return carry

                lax.fori_loop(0, nb // 2, pair, 0)
                plsc.subcore_barrier()
                dpt = size // 16
                for off in range(0, dpt, EB):
                    sz = min(EB, dpt - off)
                    pltpu.sync_copy(acc.at[pl.ds(s * dpt + off, sz)],
                                    rows0.at[pl.ds(0, sz)])
                    pltpu.sync_copy(rows0.at[pl.ds(0, sz)],
                                    out.at[pl.ds(base + s * dpt + off, sz)])

    return body


def _layer_call(chunks, nblk, na_pad, nb_pad, src, dst, ha, hb):
    body = _make_layer(chunks, nblk)
    k = pl.kernel(
        body,
        out_type=(jax.ShapeDtypeStruct((na_pad, D), jnp.float32),
                  jax.ShapeDtypeStruct((nb_pad, D), jnp.float32)),
        mesh=_mesh(),
        scratch_types=[
            pltpu.VMEM((EB,), jnp.int32),
            pltpu.VMEM((EB,), jnp.int32),
            pltpu.VMEM((2, 128), jnp.int32),
            pltpu.VMEM((EB, D), jnp.float32),
            pltpu.VMEM((EB,), jnp.int32),
            pltpu.VMEM((EB,), jnp.int32),
            pltpu.VMEM((2, 128), jnp.int32),
            pltpu.VMEM((EB, D), jnp.float32),
            pltpu.VMEM((8, D), jnp.float32),
            pltpu.VMEM_SHARED((ACC_ROWS, D), jnp.float32),
            pltpu.SemaphoreType.DMA,
            pltpu.SemaphoreType.DMA,
            pltpu.SemaphoreType.DMA,
            pltpu.SemaphoreType.DMA,
        ],
        compiler_params=pltpu.CompilerParams(use_tc_tiling_on_sc=False, needs_layout_passes=False),
    )
    return k(src, dst, ha, hb)


UI_CHUNKS = ((0, 0, CH_U, 0), (0, CH_U, CH_U, 1), (0, 2 * CH_U, CH_U, 0),
             (1, 0, CH_I, 1), (1, CH_I, CH_I, 0), (1, 2 * CH_I, CH_I, 1),
             (1, 3 * CH_I, CH_I, 0), (1, 4 * CH_I, CH_I, 1))
UB_CHUNKS = ((0, 0, CH_U, 0), (0, CH_U, CH_U, 1), (0, 2 * CH_U, CH_U, 1),
             (1, 0, PADB, 0))


def _gather_body(uix_h, bfl_h, bit_h, ilu_t, ili_t, blu_t, blb_t,
                 ilu_g, blu_g, ie_g, blb_g,
                 uix, bflv, uix32, bix64, ids2d, iix, rows, sem):
    c = lax.axis_index("c")
    s = lax.axis_index("s")
    wid = s * 2 + c
    pltpu.sync_copy(uix_h, uix)
    pltpu.sync_copy(bfl_h, bflv)
    for t in range(2):
        uix32[pl.ds(t * 16, 16)] = uix[pl.ds(wid * 32 + t * 16, 16)]
    for t in range(4):
        bix64[pl.ds(t * 16, 16)] = bflv[pl.ds(wid * 64 + t * 16, 16)]
    pltpu.async_copy(ilu_t.at[uix32], rows.at[pl.ds(0, 32)], sem).wait()
    pltpu.sync_copy(rows.at[pl.ds(0, 32)], ilu_g.at[pl.ds(wid * 32, 32)])
    pltpu.async_copy(blu_t.at[uix32], rows.at[pl.ds(0, 32)], sem).wait()
    pltpu.sync_copy(rows.at[pl.ds(0, 32)], blu_g.at[pl.ds(wid * 32, 32)])
    pltpu.async_copy(blb_t.at[bix64], rows.at[pl.ds(0, 64)], sem).wait()
    pltpu.sync_copy(rows.at[pl.ds(0, 64)], blb_g.at[pl.ds(wid * 64, 64)])
    for j in range(16):
        pltpu.async_copy(bit_h.at[bflv.at[pl.ds(j * 128, 128)]],
                         ids2d.at[pl.ds(j * 128, 128)], sem).wait()
    iota = lax.iota(jnp.int32, 16)
    for i in range(48):
        row0 = wid * 768 + i * 16
        kk = row0 // BATCH
        n = kk // S_ITEMS
        si = kk % S_ITEMS
        b0 = row0 % BATCH
        rvec = (n * BATCH + b0) + iota
        cvec = jnp.full((16,), si, jnp.int32)
        iv = plsc.load_gather(ids2d, [rvec, cvec])
        iix[pl.ds(i * 16, 16)] = iv
    for j in range(6):
        pltpu.async_copy(ili_t.at[iix.at[pl.ds(j * 128, 128)]],
                         rows.at[pl.ds(j * 128, 128)], sem).wait()
    pltpu.sync_copy(rows, ie_g.at[pl.ds(wid * 768, 768)])


def _gather_call(uix, bfl, bit, ilu_t, ili_t, blu_t, blb_t):
    k = pl.kernel(
        _gather_body,
        out_type=(jax.ShapeDtypeStruct((BATCH, D), jnp.float32),
                  jax.ShapeDtypeStruct((BATCH, D), jnp.float32),
                  jax.ShapeDtypeStruct((24576, D), jnp.float32),
                  jax.ShapeDtypeStruct((2048, D), jnp.float32)),
        mesh=_mesh(),
        scratch_types=[
            pltpu.VMEM((BATCH,), jnp.int32),
            pltpu.VMEM((2048,), jnp.int32),
            pltpu.VMEM((32,), jnp.int32),
            pltpu.VMEM((64,), jnp.int32),
            pltpu.VMEM((2048, 16), jnp.int32),
            pltpu.VMEM((768,), jnp.int32),
            pltpu.VMEM((768, D), jnp.float32),
            pltpu.SemaphoreType.DMA,
        ],
        compiler_params=pltpu.CompilerParams(use_tc_tiling_on_sc=False, needs_layout_passes=False),
    )
    return k(uix, bfl, bit, ilu_t, ili_t, blu_t, blb_t)


def _norm_body(degp_ref, out_ref):
    deg = degp_ref[pl.ds(0, DEGA)] + degp_ref[pl.ds(DEGA, DEGA)]
    out_ref[...] = 1.0 / (jnp.sqrt(deg) + 1e-8)


def _norm_call(degp):
    return pl.pallas_call(
        _norm_body,
        out_shape=jax.ShapeDtypeStruct((DEGA,), jnp.float32),
    )(degp)


def _scale_body(p, x_ref, nv_ref, o_ref):
    n = nv_ref[...]
    if p == 2:
        n = n * n
    o_ref[...] = x_ref[...] * n


def _scale(x, nv, p):
    npad = x.shape[0]
    rb = npad // 16
    return pl.pallas_call(
        functools.partial(_scale_body, p),
        grid=(16,),
        in_specs=[pl.BlockSpec((rb, D), lambda i: (i, 0)),
                  pl.BlockSpec((rb, 1), lambda i: (i, 0))],
        out_specs=pl.BlockSpec((rb, D), lambda i: (i, 0)),
        out_shape=jax.ShapeDtypeStruct((npad, D), jnp.float32),
    )(x, nv)


def _combine_body(f0_ref, s1_ref, s2_ref, nv_ref, o_ref):
    n = nv_ref[...]
    o_ref[...] = (f0_ref[...] + (s1_ref[...] + s2_ref[...]) * n) / 3.0


def _combine(f0, s1, s2, nv):
    npad = f0.shape[0]
    rb = npad // 16
    return pl.pallas_call(
        _combine_body,
        grid=(16,),
        in_specs=[pl.BlockSpec((rb, D), lambda i: (i, 0)),
                  pl.BlockSpec((rb, D), lambda i: (i, 0)),
                  pl.BlockSpec((rb, D), lambda i: (i, 0)),
                  pl.BlockSpec((rb, 1), lambda i: (i, 0))],
        out_specs=pl.BlockSpec((rb, D), lambda i: (i, 0)),
        out_shape=jax.ShapeDtypeStruct((npad, D), jnp.float32),
    )(f0, s1, s2, nv)


def _head_body(ilu_ref, blu_ref, ie_ref, blb_ref, w1_ref, b1_ref, w2_ref,
               b2_ref, out_ref):
    il_u = ilu_ref[...]
    bl_u = blu_ref[...]
    il_b = []
    for n in range(2):
        es = [ie_ref[pl.ds((n * S_ITEMS + si) * BATCH, BATCH), :]
              for si in range(S_ITEMS)]
        ls = [jnp.sum(il_u * e, axis=1) for e in es]
        m = ls[0]
        for l in ls[1:]:
            m = jnp.maximum(m, l)
        ws = [jnp.exp(l - m) for l in ls]
        z = ws[0]
        for w in ws[1:]:
            z = z + w
        acc = ws[0][:, None] * es[0]
        for w, e in zip(ws[1:], es[1:]):
            acc = acc + w[:, None] * e
        il_b.append(acc / z[:, None])

    gate_in = jnp.concatenate([bl_u, il_u], axis=1)
    h = jnp.maximum(jnp.dot(gate_in, w1_ref[...],
                            preferred_element_type=jnp.float32)
                    + b1_ref[...][None, :], 0.0)
    g = jax.nn.sigmoid(jnp.dot(h, w2_ref[...],
                               preferred_element_type=jnp.float32)
                       + b2_ref[...][None, :])
    user_rep = g * bl_u + (1.0 - g) * il_u
    preds = []
    for n in range(2):
        bl_b = blb_ref[pl.ds(n * BATCH, BATCH), :]
        bund = g * bl_b + (1.0 - g) * il_b[n]
        preds.append(jnp.sum(user_rep * bund, axis=1))
    x = preds[0] - preds[1]
    out_ref[...] = jnp.full((1, 1), jnp.mean(jnp.log1p(jnp.exp(-x))))


def _head_call(ilu, blu, ie, blb, w1, b1, w2, b2):
    return pl.pallas_call(
        _head_body,
        out_shape=jax.ShapeDtypeStruct((1, 1), jnp.float32),
    )(ilu, blu, ie, blb, w1, b1, w2, b2)


def _pad_rows(x, n_pad):
    return jnp.pad(x, ((0, n_pad - x.shape[0]), (0, 0)))


def _pad_edges(e, e_pad, base, span):
    extra = base + (jnp.arange(e_pad - e.shape[0], dtype=jnp.int32) % span)
    return jnp.concatenate([e, extra])


def kernel(users_feature, items_feature, bundles_feature, W1, b1, W2, b2,
           ui_u, ui_i, ub_u, ub_b, bundle_item_ids, bundle_item_mask,
           users, bundles):
    f0u = _pad_rows(users_feature, PADU)
    f0i = _pad_rows(items_feature, PADI)
    f0b = _pad_rows(bundles_feature, PADB)

    sui = _pad_edges(ui_u.astype(jnp.int32), EP_UI, NU, PADU - NU)
    dui = _pad_edges(ui_i.astype(jnp.int32), EP_UI, NI, PADI - NI)
    sub = _pad_edges(ub_u.astype(jnp.int32), EP_UB, NU, PADU - NU)
    dub = _pad_edges(ub_b.astype(jnp.int32), EP_UB, NBD, PADB - NBD)

    degp = _deg_call(sui, dui, sub, dub)
    norm = _norm_call(degp)
    n_ui_a = norm[OFF_UI_A:OFF_UI_A + PADU, None]
    n_ui_b = norm[OFF_UI_B:OFF_UI_B + PADI, None]
    n_ub_a = norm[OFF_UB_A:OFF_UB_A + PADU, None]
    n_ub_b = norm[OFF_UB_B:OFF_UB_B + PADB, None]

    def propagate(srcp, dstp, f0a, f0b_, nva, nvb, chunks, nblk, napad, nbpad):
        h0a = _scale(f0a, nva, 1)
        h0b = _scale(f0b_, nvb, 1)
        s1a, s1b = _layer_call(chunks, nblk, napad, nbpad, srcp, dstp,
                               h0a, h0b)
        h1a = _scale(s1a, nva, 2)
        h1b = _scale(s1b, nvb, 2)
        s2a, s2b = _layer_call(chunks, nblk, napad, nbpad, srcp, dstp,
                               h1a, h1b)
        outa = _combine(f0a, s1a, s2a, nva)
        outb = _combine(f0b_, s1b, s2b, nvb)
        return outa, outb

    il_users, il_items = propagate(sui, dui, f0u, f0i, n_ui_a, n_ui_b,
                                   UI_CHUNKS, EP_UI // EB, PADU, PADI)
    bl_users, bl_bundles = propagate(sub, dub, f0u, f0b, n_ub_a, n_ub_b,
                                     UB_CHUNKS, EP_UB // EB, PADU, PADB)

    bfl = jnp.transpose(bundles.astype(jnp.int32)).reshape(-1)
    bit16 = jnp.pad(bundle_item_ids.astype(jnp.int32), ((0, 0), (0, 4)))
    ilu_g, blu_g, ie_g, blb_g = _gather_call(
        users.astype(jnp.int32), bfl, bit16,
        il_users, il_items, bl_users, bl_bundles)

    loss = _head_call(ilu_g, blu_g, ie_g, blb_g, W1, b1, W2, b2)
    return loss.reshape(())


# speedup vs baseline: 16.7686x; 1.0315x over previous
import functools

import jax
import jax.numpy as jnp
from jax import lax
from jax.experimental import pallas as pl
from jax.experimental.pallas import tpu as pltpu
from jax.experimental.pallas import tpu_sc as plsc

D = 64
NU, NI, NBD = 50000, 100000, 20000
PADU, PADI, PADB = 50688, 100480, 20096
CH_U = 16896
CH_I = 20096
DUM = 256
ACC_ROWS = CH_I + DUM
EB = 256
E_UI, E_UB = 1_000_000, 500_000
EP_UI, EP_UB = 1_015_808, 507_904
BATCH = 1024
S_ITEMS = 12

OFF_UI_A, OFF_UI_B = 0, PADU
OFF_UB_A, OFF_UB_B = PADU + PADI, PADU + PADI + PADU
DEGTOT = PADU + PADI + PADU + PADB
DEGA = 229376

@functools.cache
def _mesh():
    return plsc.VectorSubcoreMesh(core_axis_name="c", subcore_axis_name="s",
                                  num_cores=2, num_subcores=16)


def _deg_body(sui, dui, sub, dub, degp, ones1, idxb0, idx2d0, idxb1,
              idx2d1, stage, acc, semA, semB):
    c = lax.axis_index("c")
    s = lax.axis_index("s")
    wid = s * 2 + c
    for dd in range(8):
        ones1[pl.ds(dd * 16, 16)] = jnp.ones((16,), jnp.float32)
    for dd in range(64):
        stage[pl.ds(dd * 16, 16)] = jnp.zeros((16,), jnp.float32)
    pt = DEGA // 16
    for u in range(14):
        pltpu.sync_copy(stage, acc.at[pl.ds(s * pt + u * 1024, 1024)])
    plsc.subcore_barrier()
    bufs = ((idxb0, idx2d0, semA), (idxb1, idx2d1, semB))

    def scan_list(eref, off, nblk):
        nb = nblk // 32

        def fire(i, bf):
            pltpu.async_copy(eref.at[pl.ds((i * 32 + wid) * EB, EB)],
                             bf[0], bf[2])

        def drain(bf):
            pltpu.make_async_copy(eref.at[pl.ds(0, EB)], bf[0], bf[2]).wait()

        def build(bf):
            for j in range(2):
                for t in range(8):
                    bf[1][j, pl.ds(t * 16, 16)] = (
                        bf[0][pl.ds(j * 128 + t * 16, 16)] + off)

        def scat(bf):
            for j in range(2):
                pltpu.sync_copy(ones1, acc.at[bf[1].at[j]], add=True)

        fire(0, bufs[0])
        fire(1, bufs[1])

        def pair(k2, carry):
            for half in range(2):
                i = 2 * k2 + half
                bf = bufs[half]
                drain(bf)
                build(bf)

                @pl.when(i + 2 < nb)
                def _():
                    fire(i + 2, bf)

                scat(bf)
            return carry

        lax.fori_loop(0, nb // 2, pair, 0)

    scan_list(sui, OFF_UI_A, EP_UI // EB)
    scan_list(dui, OFF_UI_B, EP_UI // EB)
    scan_list(sub, OFF_UB_A, EP_UB // EB)
    scan_list(dub, OFF_UB_B, EP_UB // EB)
    plsc.subcore_barrier()
    for u in range(14):
        pltpu.sync_copy(acc.at[pl.ds(s * pt + u * 1024, 1024)], stage)
        pltpu.sync_copy(stage, degp.at[pl.ds(c * DEGA + s * pt + u * 1024,
                                             1024)])


def _deg_call(sui, dui, sub, dub):
    k = pl.kernel(
        _deg_body,
        out_type=jax.ShapeDtypeStruct((2 * DEGA,), jnp.float32),
        mesh=_mesh(),
        scratch_types=[
            pltpu.VMEM((128,), jnp.float32),
            pltpu.VMEM((EB,), jnp.int32),
            pltpu.VMEM((2, 128), jnp.int32),
            pltpu.VMEM((EB,), jnp.int32),
            pltpu.VMEM((2, 128), jnp.int32),
            pltpu.VMEM((1024,), jnp.float32),
            pltpu.VMEM_SHARED((DEGA,), jnp.float32),
            pltpu.SemaphoreType.DMA,
            pltpu.SemaphoreType.DMA,
        ],
        compiler_params=pltpu.CompilerParams(use_tc_tiling_on_sc=False, needs_layout_passes=False),
    )
    return k(sui, dui, sub, dub)


def _make_layer(chunks, nblk):
    nb = nblk // 16

    def body(src, dst, ha, hb, sa, sb,
             sidx0, didx0, idx0, rows0, sidx1, didx1, idx1, rows1,
             zbuf, acc, semi0, semi1, semg0, semg1):
        c = lax.axis_index("c")
        s = lax.axis_index("s")
        bufs = ((sidx0, didx0, idx0, rows0, semi0, semg0),
                (sidx1, didx1, idx1, rows1, semi1, semg1))
        for dd in range(32):
            zbuf[dd // 4, pl.ds((dd % 4) * 16, 16)] = jnp.zeros((16,),
                                                               jnp.float32)

        for (side, base, size, asn) in chunks:

            @pl.when(c == asn)
            def _(side=side, base=base, size=size):
                gsrc = hb if side == 0 else ha
                out = sa if side == 0 else sb
                rpt = (size + DUM) // 16

                def zloop(u, carry):
                    pltpu.sync_copy(zbuf, acc.at[pl.ds(s * rpt + u * 8, 8)])
                    return carry

                lax.fori_loop(0, rpt // 8, zloop, 0)
                plsc.subcore_barrier()

                def fire_idx(i, bf):
                    off = (i * 16 + s) * EB
                    pltpu.async_copy(src.at[pl.ds(off, EB)], bf[0], bf[4])
                    pltpu.async_copy(dst.at[pl.ds(off, EB)], bf[1], bf[4])

                def drain_idx(bf):
                    pltpu.make_async_copy(src.at[pl.ds(0, EB)], bf[0],
                                          bf[4]).wait()
                    pltpu.make_async_copy(dst.at[pl.ds(0, EB)], bf[1],
                                          bf[4]).wait()

                def build_fire(bf):
                    ridx = bf[0] if side == 0 else bf[1]
                    gidx = bf[1] if side == 0 else bf[0]
                    for j in range(2):
                        for t in range(8):
                            rv = ridx[pl.ds(j * 128 + t * 16, 16)]
                            ok = (rv >= base) & (rv < base + size)
                            lv = jnp.where(ok, rv - base,
                                           size + (rv & (DUM - 1)))
                            bf[2][j, pl.ds(t * 16, 16)] = lv
                        pltpu.async_copy(
                            gsrc.at[gidx.at[pl.ds(j * 128, 128)]],
                            bf[3].at[pl.ds(j * 128, 128)], bf[5])

                def drain_scatter(bf):
                    pltpu.make_async_copy(gsrc.at[pl.ds(0, EB)], bf[3],
                                          bf[5]).wait()
                    for j in range(2):
                        pltpu.sync_copy(bf[3].at[pl.ds(j * 128, 128)],
                                        acc.at[bf[2].at[j]], add=True)

                fire_idx(0, bufs[0])
                fire_idx(1, bufs[1])
                drain_idx(bufs[0])
                build_fire(bufs[0])

                def pair(k2, carry):
                    for half in range(2):
                        i = 2 * k2 + half
                        bf = bufs[half]
                        ob = bufs[1 - half]

                        @pl.when(i + 1 < nb)
                        def _():
                            drain_idx(ob)
                            build_fire(ob)

                        drain_scatter(bf)

                        @pl.when(i + 2 < nb)
                        def _():
                            fire_idx(i + 2, bf)
                    return carry

                lax.fori_loop(0, nb // 2, pair, 0)
                plsc.subcore_barrier()
                dpt = size // 16
                for off in range(0, dpt, EB):
                    sz = min(EB, dpt - off)
                    pltpu.sync_copy(acc.at[pl.ds(s * dpt + off, sz)],
                                    rows0.at[pl.ds(0, sz)])
                    pltpu.sync_copy(rows0.at[pl.ds(0, sz)],
                                    out.at[pl.ds(base + s * dpt + off, sz)])

    return body


def _layer_call(chunks, nblk, na_pad, nb_pad, src, dst, ha, hb):
    body = _make_layer(chunks, nblk)
    k = pl.kernel(
        body,
        out_type=(jax.ShapeDtypeStruct((na_pad, D), jnp.float32),
                  jax.ShapeDtypeStruct((nb_pad, D), jnp.float32)),
        mesh=_mesh(),
        scratch_types=[
            pltpu.VMEM((EB,), jnp.int32),
            pltpu.VMEM((EB,), jnp.int32),
            pltpu.VMEM((2, 128), jnp.int32),
            pltpu.VMEM((EB, D), jnp.float32),
            pltpu.VMEM((EB,), jnp.int32),
            pltpu.VMEM((EB,), jnp.int32),
            pltpu.VMEM((2, 128), jnp.int32),
            pltpu.VMEM((EB, D), jnp.float32),
            pltpu.VMEM((8, D), jnp.float32),
            pltpu.VMEM_SHARED((ACC_ROWS, D), jnp.float32),
            pltpu.SemaphoreType.DMA,
            pltpu.SemaphoreType.DMA,
            pltpu.SemaphoreType.DMA,
            pltpu.SemaphoreType.DMA,
        ],
        compiler_params=pltpu.CompilerParams(use_tc_tiling_on_sc=False, needs_layout_passes=False),
    )
    return k(src, dst, ha, hb)


UI_CHUNKS = ((0, 0, CH_U, 0), (0, CH_U, CH_U, 1), (0, 2 * CH_U, CH_U, 0),
             (1, 0, CH_I, 1), (1, CH_I, CH_I, 0), (1, 2 * CH_I, CH_I, 1),
             (1, 3 * CH_I, CH_I, 0), (1, 4 * CH_I, CH_I, 1))
UB_CHUNKS = ((0, 0, CH_U, 0), (0, CH_U, CH_U, 1), (0, 2 * CH_U, CH_U, 1),
             (1, 0, PADB, 0))


def _gather_body(uix_h, bfl_h, bit_h, ilu_t, ili_t, blu_t, blb_t,
                 ilu_g, blu_g, ie_g, blb_g,
                 uix, bflv, uix32, bix64, ids2d, iix, rows, sem):
    c = lax.axis_index("c")
    s = lax.axis_index("s")
    wid = s * 2 + c
    pltpu.sync_copy(uix_h, uix)
    pltpu.sync_copy(bfl_h, bflv)
    for t in range(2):
        uix32[pl.ds(t * 16, 16)] = uix[pl.ds(wid * 32 + t * 16, 16)]
    for t in range(4):
        bix64[pl.ds(t * 16, 16)] = bflv[pl.ds(wid * 64 + t * 16, 16)]
    pltpu.async_copy(ilu_t.at[uix32], rows.at[pl.ds(0, 32)], sem).wait()
    pltpu.sync_copy(rows.at[pl.ds(0, 32)], ilu_g.at[pl.ds(wid * 32, 32)])
    pltpu.async_copy(blu_t.at[uix32], rows.at[pl.ds(0, 32)], sem).wait()
    pltpu.sync_copy(rows.at[pl.ds(0, 32)], blu_g.at[pl.ds(wid * 32, 32)])
    pltpu.async_copy(blb_t.at[bix64], rows.at[pl.ds(0, 64)], sem).wait()
    pltpu.sync_copy(rows.at[pl.ds(0, 64)], blb_g.at[pl.ds(wid * 64, 64)])
    for j in range(16):
        pltpu.async_copy(bit_h.at[bflv.at[pl.ds(j * 128, 128)]],
                         ids2d.at[pl.ds(j * 128, 128)], sem).wait()
    iota = lax.iota(jnp.int32, 16)
    for i in range(48):
        row0 = wid * 768 + i * 16
        kk = row0 // BATCH
        n = kk // S_ITEMS
        si = kk % S_ITEMS
        b0 = row0 % BATCH
        rvec = (n * BATCH + b0) + iota
        cvec = jnp.full((16,), si, jnp.int32)
        iv = plsc.load_gather(ids2d, [rvec, cvec])
        iix[pl.ds(i * 16, 16)] = iv
    for j in range(6):
        pltpu.async_copy(ili_t.at[iix.at[pl.ds(j * 128, 128)]],
                         rows.at[pl.ds(j * 128, 128)], sem).wait()
    pltpu.sync_copy(rows, ie_g.at[pl.ds(wid * 768, 768)])


def _gather_call(uix, bfl, bit, ilu_t, ili_t, blu_t, blb_t):
    k = pl.kernel(
        _gather_body,
        out_type=(jax.ShapeDtypeStruct((BATCH, D), jnp.float32),
                  jax.ShapeDtypeStruct((BATCH, D), jnp.float32),
                  jax.ShapeDtypeStruct((24576, D), jnp.float32),
                  jax.ShapeDtypeStruct((2048, D), jnp.float32)),
        mesh=_mesh(),
        scratch_types=[
            pltpu.VMEM((BATCH,), jnp.int32),
            pltpu.VMEM((2048,), jnp.int32),
            pltpu.VMEM((32,), jnp.int32),
            pltpu.VMEM((64,), jnp.int32),
            pltpu.VMEM((2048, 16), jnp.int32),
            pltpu.VMEM((768,), jnp.int32),
            pltpu.VMEM((768, D), jnp.float32),
            pltpu.SemaphoreType.DMA,
        ],
        compiler_params=pltpu.CompilerParams(use_tc_tiling_on_sc=False, needs_layout_passes=False),
    )
    return k(uix, bfl, bit, ilu_t, ili_t, blu_t, blb_t)


def _norm_body(degp_ref, out_ref):
    deg = degp_ref[pl.ds(0, DEGA)] + degp_ref[pl.ds(DEGA, DEGA)]
    out_ref[...] = 1.0 / (jnp.sqrt(deg) + 1e-8)


def _norm_call(degp):
    return pl.pallas_call(
        _norm_body,
        out_shape=jax.ShapeDtypeStruct((DEGA,), jnp.float32),
    )(degp)


def _scale_body(p, x_ref, nv_ref, o_ref):
    n = nv_ref[...]
    if p == 2:
        n = n * n
    o_ref[...] = x_ref[...] * n


def _scale(x, nv, p):
    npad = x.shape[0]
    rb = npad // 16
    return pl.pallas_call(
        functools.partial(_scale_body, p),
        grid=(16,),
        in_specs=[pl.BlockSpec((rb, D), lambda i: (i, 0)),
                  pl.BlockSpec((rb, 1), lambda i: (i, 0))],
        out_specs=pl.BlockSpec((rb, D), lambda i: (i, 0)),
        out_shape=jax.ShapeDtypeStruct((npad, D), jnp.float32),
    )(x, nv)


def _combine_body(f0_ref, s1_ref, s2_ref, nv_ref, o_ref):
    n = nv_ref[...]
    o_ref[...] = (f0_ref[...] + (s1_ref[...] + s2_ref[...]) * n) / 3.0


def _combine(f0, s1, s2, nv):
    npad = f0.shape[0]
    rb = npad // 16
    return pl.pallas_call(
        _combine_body,
        grid=(16,),
        in_specs=[pl.BlockSpec((rb, D), lambda i: (i, 0)),
                  pl.BlockSpec((rb, D), lambda i: (i, 0)),
                  pl.BlockSpec((rb, D), lambda i: (i, 0)),
                  pl.BlockSpec((rb, 1), lambda i: (i, 0))],
        out_specs=pl.BlockSpec((rb, D), lambda i: (i, 0)),
        out_shape=jax.ShapeDtypeStruct((npad, D), jnp.float32),
    )(f0, s1, s2, nv)


def _head_body(ilu_ref, blu_ref, ie_ref, blb_ref, w1_ref, b1_ref, w2_ref,
               b2_ref, out_ref):
    il_u = ilu_ref[...]
    bl_u = blu_ref[...]
    il_b = []
    for n in range(2):
        es = [ie_ref[pl.ds((n * S_ITEMS + si) * BATCH, BATCH), :]
              for si in range(S_ITEMS)]
        ls = [jnp.sum(il_u * e, axis=1) for e in es]
        m = ls[0]
        for l in ls[1:]:
            m = jnp.maximum(m, l)
        ws = [jnp.exp(l - m) for l in ls]
        z = ws[0]
        for w in ws[1:]:
            z = z + w
        acc = ws[0][:, None] * es[0]
        for w, e in zip(ws[1:], es[1:]):
            acc = acc + w[:, None] * e
        il_b.append(acc / z[:, None])

    gate_in = jnp.concatenate([bl_u, il_u], axis=1)
    h = jnp.maximum(jnp.dot(gate_in, w1_ref[...],
                            preferred_element_type=jnp.float32)
                    + b1_ref[...][None, :], 0.0)
    g = jax.nn.sigmoid(jnp.dot(h, w2_ref[...],
                               preferred_element_type=jnp.float32)
                       + b2_ref[...][None, :])
    user_rep = g * bl_u + (1.0 - g) * il_u
    preds = []
    for n in range(2):
        bl_b = blb_ref[pl.ds(n * BATCH, BATCH), :]
        bund = g * bl_b + (1.0 - g) * il_b[n]
        preds.append(jnp.sum(user_rep * bund, axis=1))
    x = preds[0] - preds[1]
    out_ref[...] = jnp.full((1, 1), jnp.mean(jnp.log1p(jnp.exp(-x))))


def _head_call(ilu, blu, ie, blb, w1, b1, w2, b2):
    return pl.pallas_call(
        _head_body,
        out_shape=jax.ShapeDtypeStruct((1, 1), jnp.float32),
    )(ilu, blu, ie, blb, w1, b1, w2, b2)


def _pad_rows(x, n_pad):
    return jnp.pad(x, ((0, n_pad - x.shape[0]), (0, 0)))


def _pad_edges(e, e_pad, base, span):
    extra = base + (jnp.arange(e_pad - e.shape[0], dtype=jnp.int32) % span)
    return jnp.concatenate([e, extra])


def kernel(users_feature, items_feature, bundles_feature, W1, b1, W2, b2,
           ui_u, ui_i, ub_u, ub_b, bundle_item_ids, bundle_item_mask,
           users, bundles):
    f0u = _pad_rows(users_feature, PADU)
    f0i = _pad_rows(items_feature, PADI)
    f0b = _pad_rows(bundles_feature, PADB)

    sui = _pad_edges(ui_u.astype(jnp.int32), EP_UI, NU, PADU - NU)
    dui = _pad_edges(ui_i.astype(jnp.int32), EP_UI, NI, PADI - NI)
    sub = _pad_edges(ub_u.astype(jnp.int32), EP_UB, NU, PADU - NU)
    dub = _pad_edges(ub_b.astype(jnp.int32), EP_UB, NBD, PADB - NBD)

    degp = _deg_call(sui, dui, sub, dub)
    norm = _norm_call(degp)
    n_ui_a = norm[OFF_UI_A:OFF_UI_A + PADU, None]
    n_ui_b = norm[OFF_UI_B:OFF_UI_B + PADI, None]
    n_ub_a = norm[OFF_UB_A:OFF_UB_A + PADU, None]
    n_ub_b = norm[OFF_UB_B:OFF_UB_B + PADB, None]

    def propagate(srcp, dstp, f0a, f0b_, nva, nvb, chunks, nblk, napad, nbpad):
        h0a = _scale(f0a, nva, 1)
        h0b = _scale(f0b_, nvb, 1)
        s1a, s1b = _layer_call(chunks, nblk, napad, nbpad, srcp, dstp,
                               h0a, h0b)
        h1a = _scale(s1a, nva, 2)
        h1b = _scale(s1b, nvb, 2)
        s2a, s2b = _layer_call(chunks, nblk, napad, nbpad, srcp, dstp,
                               h1a, h1b)
        outa = _combine(f0a, s1a, s2a, nva)
        outb = _combine(f0b_, s1b, s2b, nvb)
        return outa, outb

    il_users, il_items = propagate(sui, dui, f0u, f0i, n_ui_a, n_ui_b,
                                   UI_CHUNKS, EP_UI // EB, PADU, PADI)
    bl_users, bl_bundles = propagate(sub, dub, f0u, f0b, n_ub_a, n_ub_b,
                                     UB_CHUNKS, EP_UB // EB, PADU, PADB)

    bfl = jnp.transpose(bundles.astype(jnp.int32)).reshape(-1)
    bit16 = jnp.pad(bundle_item_ids.astype(jnp.int32), ((0, 0), (0, 4)))
    ilu_g, blu_g, ie_g, blb_g = _gather_call(
        users.astype(jnp.int32), bfl, bit16,
        il_users, il_items, bl_users, bl_bundles)

    loss = _head_call(ilu_g, blu_g, ie_g, blb_g, W1, b1, W2, b2)
    return loss.reshape(())
